# baseline (speedup 1.0000x reference)
import functools

import jax
import jax.numpy as jnp
from jax import lax
from jax.experimental import pallas as pl
from jax.experimental.pallas import tpu as pltpu
from jax.experimental.pallas import tpu_sc as plsc

_B = 16384
_E = 64
_MLP_BLK = 2048


def _sc_gather_body(nc, bpw, users_hbm, movies_hbm, eu_hbm, em_hbm,
                    u_out, m_out, idx_u, idx_m, rows_u, rows_m, sem_u, sem_m):
    wid = lax.axis_index("s") * nc + lax.axis_index("c")
    base = wid * bpw
    pltpu.sync_copy(users_hbm.at[pl.ds(base, bpw)], idx_u)
    pltpu.sync_copy(movies_hbm.at[pl.ds(base, bpw)], idx_m)
    cu = pltpu.async_copy(eu_hbm.at[idx_u], rows_u, sem_u)
    cm = pltpu.async_copy(em_hbm.at[idx_m], rows_m, sem_m)
    cu.wait()
    cm.wait()
    pltpu.sync_copy(rows_u, u_out.at[pl.ds(base, bpw)])
    pltpu.sync_copy(rows_m, m_out.at[pl.ds(base, bpw)])


def _sc_gather(users, movies, emb_users, emb_movies):
    info = plsc.get_sparse_core_info()
    nc, ns = info.num_cores, info.num_subcores
    nw = nc * ns
    bpw = _B // nw
    mesh = plsc.VectorSubcoreMesh(core_axis_name="c", subcore_axis_name="s")
    k = pl.kernel(
        functools.partial(_sc_gather_body, nc, bpw),
        out_type=(jax.ShapeDtypeStruct((_B, _E), jnp.float32),
                  jax.ShapeDtypeStruct((_B, _E), jnp.float32)),
        mesh=mesh,
        scratch_types=[
            pltpu.VMEM((bpw,), jnp.int32),
            pltpu.VMEM((bpw,), jnp.int32),
            pltpu.VMEM((bpw, _E), jnp.float32),
            pltpu.VMEM((bpw, _E), jnp.float32),
            pltpu.SemaphoreType.DMA,
            pltpu.SemaphoreType.DMA,
        ],
        compiler_params=pltpu.CompilerParams(use_tc_tiling_on_sc=False),
    )
    return k(users, movies, emb_users, emb_movies)


def _mlp_body(u_ref, m_ref, w1u_ref, w1m_ref, b1_ref, w2_ref, b2_ref,
              w3_ref, b3_ref, out_ref):
    h = jnp.dot(u_ref[...], w1u_ref[...], preferred_element_type=jnp.float32)
    h = h + jnp.dot(m_ref[...], w1m_ref[...], preferred_element_type=jnp.float32)
    h = jnp.maximum(h + b1_ref[...], 0.0)
    h = jnp.maximum(
        jnp.dot(h, w2_ref[...], preferred_element_type=jnp.float32) + b2_ref[...],
        0.0)
    o = jnp.dot(h, w3_ref[...], preferred_element_type=jnp.float32) + b3_ref[...]
    out_ref[...] = jnp.maximum(o[:, 0], 0.0)


def _mlp(u_rows, m_rows, W1, b1, W2, b2, W3, b3):
    w1u, w1m = W1[:_E], W1[_E:]
    grid = _B // _MLP_BLK
    row_spec = pl.BlockSpec((_MLP_BLK, _E), lambda i: (i, 0))

    def full(shape):
        return pl.BlockSpec(shape, lambda i: (0, 0))

    return pl.pallas_call(
        _mlp_body,
        grid=(grid,),
        in_specs=[row_spec, row_spec, full((_E, 64)), full((_E, 64)),
                  full((1, 64)), full((64, 16)), full((1, 16)),
                  full((16, 1)), full((1, 1))],
        out_specs=pl.BlockSpec((_MLP_BLK,), lambda i: (i,)),
        out_shape=jax.ShapeDtypeStruct((_B,), jnp.float32),
    )(u_rows, m_rows, w1u, w1m, b1.reshape(1, -1), W2, b2.reshape(1, -1),
      W3, b3.reshape(1, -1))


def kernel(users, movies, emb_users, emb_movies, W1, b1, W2, b2, W3, b3):
    u_rows, m_rows = _sc_gather(users.astype(jnp.int32),
                                movies.astype(jnp.int32),
                                emb_users, emb_movies)
    return _mlp(u_rows, m_rows, W1, b1, W2, b2, W3, b3)
